# baseline (speedup 1.0000x reference)
import functools

import jax
import jax.numpy as jnp
from jax import lax
from jax.experimental import pallas as pl
from jax.experimental.pallas import tpu as pltpu
from jax.experimental.pallas import tpu_sc as plsc

N_NODES = 100000
N_EDGES = 1600000
LANES = 16
IDX_W = 128
CHUNK = 400
CONV_CHUNKS = N_EDGES // (16 * CHUNK)
DCHUNK = 200
DEG_CHUNKS = N_EDGES // (32 * DCHUNK)
NBUF = 2
ZB = 136
NP = 100096
RP = NP // 8
ZERO_ROWS = NP // 16
BLKP = 736
BLKP3 = 184


def _sc_mesh():
    return plsc.VectorSubcoreMesh(core_axis_name="c", subcore_axis_name="s")


def _fill_rows(ref, n_rows, val):
    v = jnp.full((LANES,), val, jnp.float32)

    def body(i, carry):
        ref[i] = v
        return carry

    lax.fori_loop(0, n_rows, body, 0)


def _fill_flat(ref, n, val):
    v = jnp.full((LANES,), val, jnp.float32)

    def body(i, carry):
        ref[pl.ds(i * LANES, LANES)] = v
        return carry

    lax.fori_loop(0, n // LANES, body, 0)


def _zero_acc(acc, zbuf, s):
    base = s * ZERO_ROWS
    nz = zbuf.shape[0]
    done = 0
    while done < ZERO_ROWS:
        step = min(nz, ZERO_ROWS - done)
        pltpu.sync_copy(zbuf.at[pl.ds(0, step)],
                        acc.at[pl.ds(base + done, step)])
        done += step


def _deg_pass(ei):
    @functools.partial(
        pl.kernel,
        out_type=jax.ShapeDtypeStruct((2, NP, LANES), jnp.float32),
        mesh=_sc_mesh(),
        compiler_params=pltpu.CompilerParams(use_tc_tiling_on_sc=False),
        scratch_types=[
            pltpu.VMEM((2, DCHUNK), jnp.int32),
            pltpu.VMEM((DCHUNK, LANES), jnp.float32),
            pltpu.VMEM((ZB, LANES), jnp.float32),
            pltpu.VMEM_SHARED((NP, LANES), jnp.float32),
            pltpu.SemaphoreType.DMA,
            pltpu.SemaphoreType.DMA,
        ],
    )
    def kdeg(ei_h, out_h, didx, ones_v, zbuf, acc, sem0, sem1):
        c = lax.axis_index("c")
        s = lax.axis_index("s")
        sems = [sem0, sem1]
        _fill_rows(ones_v, DCHUNK, 1.0)
        _fill_rows(zbuf, ZB, 0.0)
        _zero_acc(acc, zbuf, s)
        plsc.subcore_barrier()
        wid = s * 2 + c
        base = wid * (DEG_CHUNKS * DCHUNK)

        def stage(b, i):
            pltpu.sync_copy(ei_h.at[1, pl.ds(base + i * DCHUNK, DCHUNK)],
                            didx.at[b])

        def fire(b):
            pltpu.async_copy(ones_v, acc.at[didx.at[b]], sems[b], add=True)

        def drain(b):
            pltpu.make_async_copy(
                ones_v, acc.at[didx.at[b]], sems[b]).wait()

        stage(0, 0)

        def body(t, carry):
            for k in range(2):
                i = 2 * t + k
                fire(k)

                @pl.when(i >= 1)
                def _():
                    drain(1 - k)

                @pl.when(i + 1 < DEG_CHUNKS)
                def _():
                    stage(1 - k, i + 1)
            return carry

        lax.fori_loop(0, DEG_CHUNKS // 2, body, 0)
        drain(1)
        plsc.subcore_barrier()
        sl = pl.ds(s * ZERO_ROWS, ZERO_ROWS)

        @pl.when(c == 0)
        def _():
            pltpu.sync_copy(acc.at[sl], out_h.at[0, sl])

        @pl.when(c == 1)
        def _():
            pltpu.sync_copy(acc.at[sl], out_h.at[1, sl])

    return kdeg(ei)


def _conv_pass(g, ei):
    @functools.partial(
        pl.kernel,
        out_type=jax.ShapeDtypeStruct((2, NP, LANES), jnp.float32),
        mesh=_sc_mesh(),
        compiler_params=pltpu.CompilerParams(use_tc_tiling_on_sc=False),
        scratch_types=[
            pltpu.VMEM((NBUF, CHUNK), jnp.int32),
            pltpu.VMEM((NBUF, CHUNK), jnp.int32),
            pltpu.VMEM((NBUF, CHUNK, LANES), jnp.float32),
            pltpu.VMEM((ZB, LANES), jnp.float32),
            pltpu.VMEM_SHARED((NP, LANES), jnp.float32),
        ] + [pltpu.SemaphoreType.DMA] * 4,
    )
    def kconv(g_h, ei_h, out_h, sidx, didx, rows, zbuf, acc,
              sg0, sg1, ss0, ss1):
        c = lax.axis_index("c")
        s = lax.axis_index("s")
        semg = [sg0, sg1]
        sems = [ss0, ss1]
        _fill_rows(zbuf, ZB, 0.0)
        _zero_acc(acc, zbuf, s)
        plsc.subcore_barrier()
        base = s * (CONV_CHUNKS * CHUNK)

        def stage(b, i):
            rb = pl.ds(base + i * CHUNK, CHUNK)
            pltpu.sync_copy(ei_h.at[0, rb], sidx.at[b])
            pltpu.sync_copy(ei_h.at[1, rb], didx.at[b])

        def fire_g(b):
            @pl.when(c == 0)
            def _():
                pltpu.async_copy(g_h.at[0].at[sidx.at[b]], rows.at[b],
                                 semg[b])

            @pl.when(c == 1)
            def _():
                pltpu.async_copy(g_h.at[1].at[sidx.at[b]], rows.at[b],
                                 semg[b])

        def wait_g(b):
            pltpu.make_async_copy(
                g_h.at[0].at[sidx.at[b]], rows.at[b], semg[b]).wait()

        def fire_s(b):
            pltpu.async_copy(rows.at[b], acc.at[didx.at[b]], sems[b],
                             add=True)

        def wait_s(b):
            pltpu.make_async_copy(
                rows.at[b], acc.at[didx.at[b]], sems[b]).wait()

        stage(0, 0)
        fire_g(0)

        def body(t, carry):
            for k in range(2):
                i = 2 * t + k
                wait_g(k)
                fire_s(k)

                @pl.when(i >= 1)
                def _():
                    wait_s(1 - k)

                @pl.when(i + 1 < CONV_CHUNKS)
                def _():
                    stage(1 - k, i + 1)
                    fire_g(1 - k)
            return carry

        lax.fori_loop(0, CONV_CHUNKS // 2, body, 0)
        wait_s(1)
        plsc.subcore_barrier()
        sl = pl.ds(s * ZERO_ROWS, ZERO_ROWS)

        @pl.when(c == 0)
        def _():
            pltpu.sync_copy(acc.at[sl], out_h.at[0, sl])

        @pl.when(c == 1)
        def _():
            pltpu.sync_copy(acc.at[sl], out_h.at[1, sl])

    return kconv(g, ei)


def _tc1(xp, W1bd, degp):
    def body(x_r, w_r, d_r, o_r):
        d = d_r[...]
        dinv = lax.rsqrt(d[0] + d[1] + 1.0)
        h = jnp.dot(x_r[...], w_r[...], preferred_element_type=jnp.float32)
        o_r[0] = h[:, :128] * dinv
        o_r[1] = h[:, 128:] * dinv

    return pl.pallas_call(
        body,
        grid=(RP // BLKP,),
        in_specs=[
            pl.BlockSpec((BLKP, 128), lambda i: (i, 0)),
            pl.BlockSpec(W1bd.shape, lambda i: (0, 0)),
            pl.BlockSpec((2, BLKP, 128), lambda i: (0, i, 0)),
        ],
        out_specs=pl.BlockSpec((2, BLKP, 128), lambda i: (0, i, 0)),
        out_shape=jax.ShapeDtypeStruct((2, RP, 128), jnp.float32),
    )(xp, W1bd, degp)


def _tc2(degp, acc1, g1, b1p, W2bd):
    def body(d_r, a_r, g_r, b_r, w_r, o_r):
        d = d_r[...]
        dinv = lax.rsqrt(d[0] + d[1] + 1.0)
        b = b_r[...]
        ra = jnp.maximum((a_r[0] + g_r[0]) * dinv + b[:, :128], 0.0)
        rb = jnp.maximum((a_r[1] + g_r[1]) * dinv + b[:, 128:], 0.0)
        h = jnp.dot(jnp.concatenate([ra, rb], axis=1), w_r[...],
                    preferred_element_type=jnp.float32)
        o_r[0] = h[:, :128] * dinv
        o_r[1] = h[:, 128:] * dinv

    blk2 = pl.BlockSpec((2, BLKP, 128), lambda i: (0, i, 0))
    return pl.pallas_call(
        body,
        grid=(RP // BLKP,),
        in_specs=[
            blk2, blk2, blk2,
            pl.BlockSpec(b1p.shape, lambda i: (0, 0)),
            pl.BlockSpec(W2bd.shape, lambda i: (0, 0)),
        ],
        out_specs=blk2,
        out_shape=jax.ShapeDtypeStruct((2, RP, 128), jnp.float32),
    )(degp, acc1, g1, b1p, W2bd)


def _tc3(degp, acc2, g2, b2p, xp, WLcat, bL1, WL2, bL2):
    def body(d_r, a_r, g_r, b_r, x_r, wc_r, bl1_r, wl2_r, bl2_r, o_r):
        d = d_r[...]
        dinv = lax.rsqrt(d[0] + d[1] + 1.0)
        b = b_r[...]
        ha = jnp.maximum((a_r[0] + g_r[0]) * dinv + b[:, :128], 0.0)
        hb = jnp.maximum((a_r[1] + g_r[1]) * dinv + b[:, 128:], 0.0)
        xT = jnp.transpose(x_r[...])
        haT = jnp.transpose(ha)
        hbT = jnp.transpose(hb)
        dn = (((0,), (0,)), ((), ()))
        for j in range(8):
            rs = slice(LANES * j, LANES * (j + 1))
            lhs = jnp.concatenate([xT[rs], haT[rs], hbT[rs]], axis=0)
            z = lax.dot_general(lhs, wc_r[...], dn,
                                preferred_element_type=jnp.float32)
            z = jnp.maximum(z + bl1_r[...], 0.0)
            o = jnp.dot(z, wl2_r[...], preferred_element_type=jnp.float32)
            o_r[:, pl.ds(j, 1)] = jax.nn.sigmoid(o + bl2_r[...])

    blk2 = pl.BlockSpec((2, BLKP3, 128), lambda i: (0, i, 0))
    full = lambda a: pl.BlockSpec(a.shape, lambda i: (0, 0))
    return pl.pallas_call(
        body,
        grid=(RP // BLKP3,),
        in_specs=[
            blk2, blk2, blk2,
            pl.BlockSpec(b2p.shape, lambda i: (0, 0)),
            pl.BlockSpec((BLKP3, 128), lambda i: (i, 0)),
            full(WLcat), full(bL1), full(WL2), full(bL2),
        ],
        out_specs=pl.BlockSpec((BLKP3, 8), lambda i: (i, 0)),
        out_shape=jax.ShapeDtypeStruct((RP, 8), jnp.float32),
    )(degp, acc2, g2, b2p, xp, WLcat, bL1, WL2, bL2)


def _block_diag8(w):
    k = w.shape[1]
    out = jnp.zeros((128, 8 * k), jnp.float32)
    for j in range(8):
        out = out.at[16 * j:16 * (j + 1), k * j:k * (j + 1)].set(w)
    return out


def kernel(x, edge_index, W1, b1, W2, b2, WL1, bL1, WL2, bL2):
    n_nodes = x.shape[0]
    in_ch = x.shape[1]
    assert n_nodes == N_NODES and edge_index.shape[1] == N_EDGES

    ei = edge_index.astype(jnp.int32)

    xpad = jnp.zeros((NP, LANES), jnp.float32).at[:N_NODES, :in_ch].set(x)
    xp = xpad.reshape(RP, 128)

    W1p = jnp.zeros((LANES, 32), jnp.float32).at[:in_ch].set(W1)
    W1bd = jnp.concatenate(
        [_block_diag8(W1p[:, :16]), _block_diag8(W1p[:, 16:])], axis=1)
    W2bd = jnp.block(
        [[_block_diag8(W2[:16, :16]), _block_diag8(W2[:16, 16:])],
         [_block_diag8(W2[16:, :16]), _block_diag8(W2[16:, 16:])]])
    b1p = jnp.concatenate([jnp.tile(b1[:16], 8), jnp.tile(b1[16:], 8)])
    b1p = b1p.reshape(1, 256)
    b2p = jnp.concatenate([jnp.tile(b2[:16], 8), jnp.tile(b2[16:], 8)])
    b2p = b2p.reshape(1, 256)
    WL1x = jnp.zeros((LANES, 1024), jnp.float32).at[:in_ch].set(WL1[:in_ch])
    WLcat = jnp.concatenate(
        [WL1x, WL1[in_ch:in_ch + 16], WL1[in_ch + 16:in_ch + 32]], axis=0)
    bL1r = bL1.reshape(1, -1)
    bL2r = bL2.reshape(1, -1)

    degp = _deg_pass(ei).reshape(2, RP, 128)

    g1 = _tc1(xp, W1bd, degp)
    acc1 = _conv_pass(g1.reshape(2, NP, LANES), ei).reshape(2, RP, 128)

    g2 = _tc2(degp, acc1, g1, b1p, W2bd)
    acc2 = _conv_pass(g2.reshape(2, NP, LANES), ei).reshape(2, RP, 128)

    out = _tc3(degp, acc2, g2, b2p, xp, WLcat, bL1r, WL2, bL2r)
    return out.reshape(NP, 1)[:N_NODES]


# speedup vs baseline: 1.3647x; 1.3647x over previous
import functools

import jax
import jax.numpy as jnp
from jax import lax
from jax.experimental import pallas as pl
from jax.experimental.pallas import tpu as pltpu
from jax.experimental.pallas import tpu_sc as plsc

N_NODES = 100000
N_EDGES = 1600000
LANES = 16
IDX_W = 128
K_PER_CHUNK = 6
CHUNK = IDX_W * K_PER_CHUNK
EDGES_PAD = 132 * 16 * CHUNK
IDX_ROWS = EDGES_PAD // IDX_W
CONV_CHUNKS = EDGES_PAD // (16 * CHUNK)
DEG_CHUNKS = EDGES_PAD // (32 * CHUNK)
NBUF = 2
ZB = 136
NP = 100096
RP = NP // 8
ZERO_ROWS = NP // 16
BLKP = 736
BLKP3 = 184


def _sc_mesh():
    return plsc.VectorSubcoreMesh(core_axis_name="c", subcore_axis_name="s")


def _fill_rows(ref, n_rows, val):
    v = jnp.full((LANES,), val, jnp.float32)

    def body(i, carry):
        ref[i] = v
        return carry

    lax.fori_loop(0, n_rows, body, 0)


def _zero_acc(acc, zbuf, s):
    base = s * ZERO_ROWS
    nz = zbuf.shape[0]
    done = 0
    while done < ZERO_ROWS:
        step = min(nz, ZERO_ROWS - done)
        pltpu.sync_copy(zbuf.at[pl.ds(0, step)],
                        acc.at[pl.ds(base + done, step)])
        done += step


def _deg_pass(ei):
    @functools.partial(
        pl.kernel,
        out_type=jax.ShapeDtypeStruct((2, NP, LANES), jnp.float32),
        mesh=_sc_mesh(),
        compiler_params=pltpu.CompilerParams(use_tc_tiling_on_sc=False),
        scratch_types=[
            pltpu.VMEM((2, K_PER_CHUNK, IDX_W), jnp.int32),
            pltpu.VMEM((IDX_W, LANES), jnp.float32),
            pltpu.VMEM((ZB, LANES), jnp.float32),
            pltpu.VMEM_SHARED((NP, LANES), jnp.float32),
            pltpu.SemaphoreType.DMA,
            pltpu.SemaphoreType.DMA,
        ],
    )
    def kdeg(ei_h, out_h, didx, ones_v, zbuf, acc, sem0, sem1):
        c = lax.axis_index("c")
        s = lax.axis_index("s")
        sems = [sem0, sem1]
        _fill_rows(ones_v, IDX_W, 1.0)
        _fill_rows(zbuf, ZB, 0.0)
        _zero_acc(acc, zbuf, s)
        plsc.subcore_barrier()
        wid = s * 2 + c
        base = wid * (DEG_CHUNKS * K_PER_CHUNK)

        def stage(b, i):
            rb = base + i * K_PER_CHUNK
            pltpu.sync_copy(ei_h.at[1, pl.ds(rb, K_PER_CHUNK)], didx.at[b])

        def fire(b):
            for j in range(K_PER_CHUNK):
                pltpu.async_copy(ones_v, acc.at[didx.at[b, j]], sems[b],
                                 add=True)

        def drain(b):
            pltpu.make_async_copy(
                ones_v, acc.at[didx.at[b, 0]], sems[b]).wait()

        stage(0, 0)

        def body(t, carry):
            for k in range(2):
                i = 2 * t + k
                fire(k)

                @pl.when(i >= 1)
                def _():
                    for _j in range(K_PER_CHUNK):
                        drain(1 - k)

                @pl.when(i + 1 < DEG_CHUNKS)
                def _():
                    stage(1 - k, i + 1)
            return carry

        lax.fori_loop(0, DEG_CHUNKS // 2, body, 0)
        for _j in range(K_PER_CHUNK):
            drain(1)
        plsc.subcore_barrier()
        sl = pl.ds(s * ZERO_ROWS, ZERO_ROWS)

        @pl.when(c == 0)
        def _():
            pltpu.sync_copy(acc.at[sl], out_h.at[0, sl])

        @pl.when(c == 1)
        def _():
            pltpu.sync_copy(acc.at[sl], out_h.at[1, sl])

    return kdeg(ei)


def _conv_pass(g, ei):
    @functools.partial(
        pl.kernel,
        out_type=jax.ShapeDtypeStruct((2, NP, LANES), jnp.float32),
        mesh=_sc_mesh(),
        compiler_params=pltpu.CompilerParams(use_tc_tiling_on_sc=False),
        scratch_types=[
            pltpu.VMEM((NBUF, K_PER_CHUNK, IDX_W), jnp.int32),
            pltpu.VMEM((NBUF, K_PER_CHUNK, IDX_W), jnp.int32),
            pltpu.VMEM((NBUF, CHUNK, LANES), jnp.float32),
            pltpu.VMEM((ZB, LANES), jnp.float32),
            pltpu.VMEM_SHARED((NP, LANES), jnp.float32),
        ] + [pltpu.SemaphoreType.DMA] * 4,
    )
    def kconv(g_h, ei_h, out_h, sidx, didx, rows, zbuf, acc,
              sg0, sg1, ss0, ss1):
        c = lax.axis_index("c")
        s = lax.axis_index("s")
        semg = [sg0, sg1]
        sems = [ss0, ss1]
        _fill_rows(zbuf, ZB, 0.0)
        _zero_acc(acc, zbuf, s)
        plsc.subcore_barrier()
        base = s * (CONV_CHUNKS * K_PER_CHUNK)

        def stage(b, i):
            rb = base + i * K_PER_CHUNK
            pltpu.sync_copy(ei_h.at[0, pl.ds(rb, K_PER_CHUNK)], sidx.at[b])
            pltpu.sync_copy(ei_h.at[1, pl.ds(rb, K_PER_CHUNK)], didx.at[b])

        def fire_g(b):
            @pl.when(c == 0)
            def _():
                for j in range(K_PER_CHUNK):
                    pltpu.async_copy(g_h.at[0].at[sidx.at[b, j]],
                                     rows.at[b, pl.ds(j * IDX_W, IDX_W)],
                                     semg[b])

            @pl.when(c == 1)
            def _():
                for j in range(K_PER_CHUNK):
                    pltpu.async_copy(g_h.at[1].at[sidx.at[b, j]],
                                     rows.at[b, pl.ds(j * IDX_W, IDX_W)],
                                     semg[b])

        def wait_g(b):
            pltpu.make_async_copy(
                g_h.at[0].at[sidx.at[b, 0]], rows.at[b], semg[b]).wait()

        def fire_s(b):
            for j in range(K_PER_CHUNK):
                pltpu.async_copy(rows.at[b, pl.ds(j * IDX_W, IDX_W)],
                                 acc.at[didx.at[b, j]], sems[b], add=True)

        def wait_s(b):
            pltpu.make_async_copy(
                rows.at[b], acc.at[didx.at[b, 0]], sems[b]).wait()

        stage(0, 0)
        fire_g(0)

        def body(t, carry):
            for k in range(2):
                i = 2 * t + k
                wait_g(k)
                fire_s(k)

                @pl.when(i >= 1)
                def _():
                    wait_s(1 - k)

                @pl.when(i + 1 < CONV_CHUNKS)
                def _():
                    stage(1 - k, i + 1)
                    fire_g(1 - k)
            return carry

        lax.fori_loop(0, CONV_CHUNKS // 2, body, 0)
        wait_s(1)
        plsc.subcore_barrier()
        sl = pl.ds(s * ZERO_ROWS, ZERO_ROWS)

        @pl.when(c == 0)
        def _():
            pltpu.sync_copy(acc.at[sl], out_h.at[0, sl])

        @pl.when(c == 1)
        def _():
            pltpu.sync_copy(acc.at[sl], out_h.at[1, sl])

    return kconv(g, ei)


def _tc1(xp, W1bd, degp):
    def body(x_r, w_r, d_r, o_r):
        d = d_r[...]
        dinv = lax.rsqrt(d[0] + d[1] + 1.0)
        h = jnp.dot(x_r[...], w_r[...], preferred_element_type=jnp.float32)
        o_r[0] = h[:, :128] * dinv
        o_r[1] = h[:, 128:] * dinv

    return pl.pallas_call(
        body,
        grid=(RP // BLKP,),
        in_specs=[
            pl.BlockSpec((BLKP, 128), lambda i: (i, 0)),
            pl.BlockSpec(W1bd.shape, lambda i: (0, 0)),
            pl.BlockSpec((2, BLKP, 128), lambda i: (0, i, 0)),
        ],
        out_specs=pl.BlockSpec((2, BLKP, 128), lambda i: (0, i, 0)),
        out_shape=jax.ShapeDtypeStruct((2, RP, 128), jnp.float32),
    )(xp, W1bd, degp)


def _tc2(degp, acc1, g1, b1p, W2bd):
    def body(d_r, a_r, g_r, b_r, w_r, o_r):
        d = d_r[...]
        dinv = lax.rsqrt(d[0] + d[1] + 1.0)
        b = b_r[...]
        ra = jnp.maximum((a_r[0] + g_r[0]) * dinv + b[:, :128], 0.0)
        rb = jnp.maximum((a_r[1] + g_r[1]) * dinv + b[:, 128:], 0.0)
        h = jnp.dot(jnp.concatenate([ra, rb], axis=1), w_r[...],
                    preferred_element_type=jnp.float32)
        o_r[0] = h[:, :128] * dinv
        o_r[1] = h[:, 128:] * dinv

    blk2 = pl.BlockSpec((2, BLKP, 128), lambda i: (0, i, 0))
    return pl.pallas_call(
        body,
        grid=(RP // BLKP,),
        in_specs=[
            blk2, blk2, blk2,
            pl.BlockSpec(b1p.shape, lambda i: (0, 0)),
            pl.BlockSpec(W2bd.shape, lambda i: (0, 0)),
        ],
        out_specs=blk2,
        out_shape=jax.ShapeDtypeStruct((2, RP, 128), jnp.float32),
    )(degp, acc1, g1, b1p, W2bd)


def _tc3(degp, acc2, g2, b2p, xp, WLcat, bL1, WL2, bL2):
    def body(d_r, a_r, g_r, b_r, x_r, wc_r, bl1_r, wl2_r, bl2_r, o_r):
        d = d_r[...]
        dinv = lax.rsqrt(d[0] + d[1] + 1.0)
        b = b_r[...]
        ha = jnp.maximum((a_r[0] + g_r[0]) * dinv + b[:, :128], 0.0)
        hb = jnp.maximum((a_r[1] + g_r[1]) * dinv + b[:, 128:], 0.0)
        xT = jnp.transpose(x_r[...])
        haT = jnp.transpose(ha)
        hbT = jnp.transpose(hb)
        dn = (((0,), (0,)), ((), ()))
        for j in range(8):
            rs = slice(LANES * j, LANES * (j + 1))
            lhs = jnp.concatenate([xT[rs], haT[rs], hbT[rs]], axis=0)
            z = lax.dot_general(lhs, wc_r[...], dn,
                                preferred_element_type=jnp.float32)
            z = jnp.maximum(z + bl1_r[...], 0.0)
            o = jnp.dot(z, wl2_r[...], preferred_element_type=jnp.float32)
            o_r[:, pl.ds(j, 1)] = jax.nn.sigmoid(o + bl2_r[...])

    blk2 = pl.BlockSpec((2, BLKP3, 128), lambda i: (0, i, 0))
    full = lambda a: pl.BlockSpec(a.shape, lambda i: (0, 0))
    return pl.pallas_call(
        body,
        grid=(RP // BLKP3,),
        in_specs=[
            blk2, blk2, blk2,
            pl.BlockSpec(b2p.shape, lambda i: (0, 0)),
            pl.BlockSpec((BLKP3, 128), lambda i: (i, 0)),
            full(WLcat), full(bL1), full(WL2), full(bL2),
        ],
        out_specs=pl.BlockSpec((BLKP3, 8), lambda i: (i, 0)),
        out_shape=jax.ShapeDtypeStruct((RP, 8), jnp.float32),
    )(degp, acc2, g2, b2p, xp, WLcat, bL1, WL2, bL2)


def _block_diag8(w):
    k = w.shape[1]
    out = jnp.zeros((128, 8 * k), jnp.float32)
    for j in range(8):
        out = out.at[16 * j:16 * (j + 1), k * j:k * (j + 1)].set(w)
    return out


def kernel(x, edge_index, W1, b1, W2, b2, WL1, bL1, WL2, bL2):
    n_nodes = x.shape[0]
    in_ch = x.shape[1]
    assert n_nodes == N_NODES and edge_index.shape[1] == N_EDGES

    pad = EDGES_PAD - N_EDGES
    pad_src = (jnp.arange(pad, dtype=jnp.int32) * 17) % N_NODES
    pad_dst = N_NODES + (jnp.arange(pad, dtype=jnp.int32) % (NP - N_NODES))
    ei = jnp.concatenate(
        [edge_index.astype(jnp.int32),
         jnp.stack([pad_src, pad_dst])], axis=1).reshape(2, IDX_ROWS, IDX_W)

    xpad = jnp.zeros((NP, LANES), jnp.float32).at[:N_NODES, :in_ch].set(x)
    xp = xpad.reshape(RP, 128)

    W1p = jnp.zeros((LANES, 32), jnp.float32).at[:in_ch].set(W1)
    W1bd = jnp.concatenate(
        [_block_diag8(W1p[:, :16]), _block_diag8(W1p[:, 16:])], axis=1)
    W2bd = jnp.block(
        [[_block_diag8(W2[:16, :16]), _block_diag8(W2[:16, 16:])],
         [_block_diag8(W2[16:, :16]), _block_diag8(W2[16:, 16:])]])
    b1p = jnp.concatenate([jnp.tile(b1[:16], 8), jnp.tile(b1[16:], 8)])
    b1p = b1p.reshape(1, 256)
    b2p = jnp.concatenate([jnp.tile(b2[:16], 8), jnp.tile(b2[16:], 8)])
    b2p = b2p.reshape(1, 256)
    WL1x = jnp.zeros((LANES, 1024), jnp.float32).at[:in_ch].set(WL1[:in_ch])
    WLcat = jnp.concatenate(
        [WL1x, WL1[in_ch:in_ch + 16], WL1[in_ch + 16:in_ch + 32]], axis=0)
    bL1r = bL1.reshape(1, -1)
    bL2r = bL2.reshape(1, -1)

    degp = _deg_pass(ei).reshape(2, RP, 128)

    g1 = _tc1(xp, W1bd, degp)
    acc1 = _conv_pass(g1.reshape(2, NP, LANES), ei).reshape(2, RP, 128)

    g2 = _tc2(degp, acc1, g1, b1p, W2bd)
    acc2 = _conv_pass(g2.reshape(2, NP, LANES), ei).reshape(2, RP, 128)

    out = _tc3(degp, acc2, g2, b2p, xp, WLcat, bL1r, WL2, bL2r)
    return out.reshape(NP, 1)[:N_NODES]
